# baseline (speedup 1.0000x reference)
import functools

import jax
import jax.numpy as jnp
from jax import lax
from jax.experimental import pallas as pl
from jax.experimental.pallas import tpu as pltpu
from jax.experimental.pallas import tpu_sc as plsc

_B = 1024
_K = 1024
_D = 64
_BB = 32


def _argmin_kernel(x_ref, lmt_ref, idx_ref):
    x = x_ref[...]
    lmt = lmt_ref[...]
    xlm = jax.lax.dot_general(
        x, lmt, (((1,), (0,)), ((), ())), preferred_element_type=jnp.float32
    )
    x2 = jnp.sum(x * x, axis=1, keepdims=True)
    lm2 = jnp.sum(lmt * lmt, axis=0, keepdims=True)
    dist = x2 + lm2 - 2.0 * xlm
    dmin = jnp.min(dist, axis=1, keepdims=True)
    iota = jax.lax.broadcasted_iota(jnp.int32, dist.shape, 1)
    idx = jnp.min(jnp.where(dist == dmin, iota, _K), axis=1, keepdims=True)
    idx_ref[...] = idx


def _delta_kernel(x_ref, lmt_ref, h_ref, out_ref):
    x = x_ref[...]
    lmt = lmt_ref[...]
    h = h_ref[...]
    out_ref[...] = h[:, None, :] * (x[:, :, None] - lmt[None, :, :])


def _make_sc_gather():
    info = plsc.get_sparse_core_info()
    nw = info.num_cores * info.num_subcores
    b_per_w = _B // nw
    mesh = plsc.VectorSubcoreMesh(core_axis_name="c", subcore_axis_name="s")

    @functools.partial(
        pl.kernel, mesh=mesh,
        out_type=jax.ShapeDtypeStruct((_B, _K), jnp.float32),
        scratch_types=[
            pltpu.VMEM((b_per_w,), jnp.int32),
            pltpu.VMEM((b_per_w, _K), jnp.float32),
            pltpu.SemaphoreType.DMA,
        ],
    )
    def gather_k(table_hbm, idx_hbm, out_hbm, idx_v, rows_v, sem):
        wid = lax.axis_index("s") * info.num_cores + lax.axis_index("c")
        base = wid * b_per_w
        pltpu.sync_copy(idx_hbm.at[pl.ds(base, b_per_w)], idx_v)
        pltpu.async_copy(table_hbm.at[idx_v], rows_v, sem).wait()
        pltpu.sync_copy(rows_v, out_hbm.at[pl.ds(base, b_per_w)])

    return gather_k


_sc_gather = _make_sc_gather()


@jax.jit
def kernel(x, landmarks, qd):
    lmt = landmarks.T
    idx2d = pl.pallas_call(
        _argmin_kernel,
        grid=(1,),
        in_specs=[
            pl.BlockSpec((_B, _D), lambda i: (0, 0)),
            pl.BlockSpec((_D, _K), lambda i: (0, 0)),
        ],
        out_specs=pl.BlockSpec((_B, 1), lambda i: (0, 0)),
        out_shape=jax.ShapeDtypeStruct((_B, 1), jnp.int32),
    )(x, lmt)
    idx = idx2d.reshape(_B)
    h = _sc_gather(qd, idx)
    out_t = pl.pallas_call(
        _delta_kernel,
        grid=(_B // _BB,),
        in_specs=[
            pl.BlockSpec((_BB, _D), lambda i: (i, 0)),
            pl.BlockSpec((_D, _K), lambda i: (0, 0)),
            pl.BlockSpec((_BB, _K), lambda i: (i, 0)),
        ],
        out_specs=pl.BlockSpec((_BB, _D, _K), lambda i: (i, 0, 0)),
        out_shape=jax.ShapeDtypeStruct((_B, _D, _K), jnp.float32),
        compiler_params=pltpu.CompilerParams(
            dimension_semantics=("parallel",),
        ),
    )(x, lmt, h)
    return jnp.swapaxes(out_t, 1, 2)


# speedup vs baseline: 1.2822x; 1.2822x over previous
import jax
import jax.numpy as jnp
from jax.experimental import pallas as pl
from jax.experimental.pallas import tpu as pltpu

_B = 1024
_K = 1024
_D = 64
_BB = 32
_NBLK = _B // _BB
_NBUF = 4


def _som_kernel(x_ref, lmt_ref, qd_ref, out_hbm, scratch, sems):
    i = pl.program_id(0)
    s = jax.lax.rem(i, _NBUF)

    @pl.when(i >= _NBUF)
    def _wait_reuse():
        old = (i - _NBUF) * _BB
        pltpu.make_async_copy(
            scratch.at[s], out_hbm.at[pl.ds(old, _BB)], sems.at[s]
        ).wait()

    x = x_ref[...]
    lmt = lmt_ref[...]
    xlm = jax.lax.dot_general(
        x, lmt, (((1,), (0,)), ((), ())), preferred_element_type=jnp.float32
    )
    x2 = jnp.sum(x * x, axis=1, keepdims=True)
    lm2 = jnp.sum(lmt * lmt, axis=0, keepdims=True)
    dist = x2 + lm2 - 2.0 * xlm
    dmin = jnp.min(dist, axis=1, keepdims=True)
    iota = jax.lax.broadcasted_iota(jnp.int32, dist.shape, 1)
    idx = jnp.min(jnp.where(dist == dmin, iota, _K), axis=1, keepdims=True)
    onehot = (iota == idx).astype(jnp.float32)
    h = jax.lax.dot_general(
        onehot, qd_ref[...], (((1,), (0,)), ((), ())),
        preferred_element_type=jnp.float32,
    )
    scratch[s] = h[:, None, :] * (x[:, :, None] - lmt[None, :, :])
    pltpu.make_async_copy(
        scratch.at[s], out_hbm.at[pl.ds(i * _BB, _BB)], sems.at[s]
    ).start()

    @pl.when(i == _NBLK - 1)
    def _drain():
        for j in range(_NBUF):
            step = _NBLK - _NBUF + j
            pltpu.make_async_copy(
                scratch.at[j], out_hbm.at[pl.ds(step * _BB, _BB)], sems.at[j]
            ).wait()


@jax.jit
def kernel(x, landmarks, qd):
    out_t = pl.pallas_call(
        _som_kernel,
        grid=(_NBLK,),
        in_specs=[
            pl.BlockSpec((_BB, _D), lambda i: (i, 0)),
            pl.BlockSpec((_D, _K), lambda i: (0, 0)),
            pl.BlockSpec((_K, _K), lambda i: (0, 0)),
        ],
        out_specs=pl.BlockSpec(memory_space=pl.ANY),
        out_shape=jax.ShapeDtypeStruct((_B, _D, _K), jnp.float32),
        scratch_shapes=[
            pltpu.VMEM((_NBUF, _BB, _D, _K), jnp.float32),
            pltpu.SemaphoreType.DMA((_NBUF,)),
        ],
    )(x, landmarks.T, qd)
    return jnp.swapaxes(out_t, 1, 2)
